# Initial kernel scaffold; baseline (speedup 1.0000x reference)
import functools

import jax
import jax.numpy as jnp
from jax import lax
from jax.experimental import pallas as pl
from jax.experimental.pallas import tpu as pltpu
from jax.experimental.pallas import tpu_sc as plsc

N_NODES = 10000
E_EDGES = 320000
D = 128

NC = 2
NS = 16
NW = NC * NS
CH = 80
ROWS2D = E_EDGES // CH
NCHUNK = ROWS2D // NW
NPT = N_NODES // NS

_MESH = plsc.VectorSubcoreMesh(core_axis_name="c", subcore_axis_name="s")


def _deg_body(dst_hbm, ew_hbm, out_hbm, dst_v, ew_v, rows_v, zer_v, acc_s, sem):
    cid = lax.axis_index("c")
    sid = lax.axis_index("s")
    wid = cid * NS + sid
    base = wid * NCHUNK
    pltpu.sync_copy(dst_hbm.at[pl.ds(base, NCHUNK)], dst_v)
    pltpu.sync_copy(ew_hbm.at[pl.ds(base, NCHUNK)], ew_v)

    zv = jnp.zeros((16,), jnp.float32)

    def zrow(i, _):
        zer_v[i, :] = zv
        return ()

    lax.fori_loop(0, NPT, zrow, ())
    for e in range(CH):
        rows_v[e, :] = zv
    nb = sid * NPT
    pltpu.sync_copy(zer_v, acc_s.at[pl.ds(nb, NPT)])
    plsc.subcore_barrier()

    lane0 = jnp.zeros((16,), jnp.int32)
    iot = lax.iota(jnp.int32, 16)

    def chunk(c, _):
        for g in range(CH // 16):
            ew16 = ew_v[c, pl.ds(g * 16, 16)]
            plsc.store_scatter(rows_v, [iot + (g * 16), lane0], ew16)
        pltpu.sync_copy(rows_v, acc_s.at[dst_v.at[c]], add=True)
        return ()

    lax.fori_loop(0, NCHUNK, chunk, ())
    plsc.subcore_barrier()
    pltpu.sync_copy(acc_s.at[pl.ds(nb, NPT)], out_hbm.at[cid, pl.ds(nb, NPT)])


@jax.jit
def _deg_call(dst2, ew2):
    return pl.kernel(
        _deg_body,
        out_type=jax.ShapeDtypeStruct((NC, N_NODES, 16), jnp.float32),
        mesh=_MESH,
        scratch_types=[
            pltpu.VMEM((NCHUNK, CH), jnp.int32),
            pltpu.VMEM((NCHUNK, CH), jnp.float32),
            pltpu.VMEM((CH, 16), jnp.float32),
            pltpu.VMEM((NPT, 16), jnp.float32),
            pltpu.VMEM_SHARED((N_NODES, 16), jnp.float32),
            pltpu.SemaphoreType.DMA,
        ],
    )(dst2, ew2)


def _edge_body(hs_hbm, src_hbm, dst_hbm, ew_hbm, out_hbm,
               src_v, dst_v, ew_v, rows_v, acc_s, sem):
    cid = lax.axis_index("c")
    sid = lax.axis_index("s")
    wid = cid * NS + sid
    base = wid * NCHUNK
    pltpu.sync_copy(src_hbm.at[pl.ds(base, NCHUNK)], src_v)
    pltpu.sync_copy(dst_hbm.at[pl.ds(base, NCHUNK)], dst_v)
    pltpu.sync_copy(ew_hbm.at[pl.ds(base, NCHUNK)], ew_v)
    nb = sid * NPT
    pltpu.sync_copy(hs_hbm.at[pl.ds(nb, NPT)], acc_s.at[pl.ds(nb, NPT)])
    plsc.subcore_barrier()

    def chunk(c, _):
        pltpu.async_copy(hs_hbm.at[src_v.at[c]], rows_v, sem).wait()
        for g in range(CH // 16):
            ew16 = ew_v[c, pl.ds(g * 16, 16)]
            for l in range(16):
                e = g * 16 + l
                bc = jnp.take(ew16, jnp.full((16,), l, jnp.int32),
                              mode="promise_in_bounds")
                for j in range(D // 16):
                    rows_v[e, pl.ds(j * 16, 16)] = (
                        rows_v[e, pl.ds(j * 16, 16)] * bc)
        pltpu.sync_copy(rows_v, acc_s.at[dst_v.at[c]], add=True)
        return ()

    lax.fori_loop(0, NCHUNK, chunk, ())
    plsc.subcore_barrier()
    pltpu.sync_copy(acc_s.at[pl.ds(nb, NPT)], out_hbm.at[cid, pl.ds(nb, NPT)])


@jax.jit
def _edge_call(hs, src2, dst2, ew2):
    return pl.kernel(
        _edge_body,
        out_type=jax.ShapeDtypeStruct((NC, N_NODES, D), jnp.float32),
        mesh=_MESH,
        scratch_types=[
            pltpu.VMEM((NCHUNK, CH), jnp.int32),
            pltpu.VMEM((NCHUNK, CH), jnp.int32),
            pltpu.VMEM((NCHUNK, CH), jnp.float32),
            pltpu.VMEM((CH, D), jnp.float32),
            pltpu.VMEM_SHARED((N_NODES, D), jnp.float32),
            pltpu.SemaphoreType.DMA,
        ],
    )(hs, src2, dst2, ew2)


RB = 1000


def _tc1_body(x_ref, w_ref, degp_ref, hs_ref, dis_ref):
    deg = jnp.sum(degp_ref[...], axis=(0, 2)) + 1.0
    dis = jnp.where(deg > 0.0, lax.rsqrt(deg), 0.0)
    h = jnp.dot(x_ref[...], w_ref[...], preferred_element_type=jnp.float32)
    hs_ref[...] = h * dis[:, None]
    dis_ref[...] = dis


@jax.jit
def _tc1_call(x, W1, degp):
    grid = (N_NODES // RB,)
    return pl.pallas_call(
        _tc1_body,
        grid=grid,
        in_specs=[
            pl.BlockSpec((RB, D), lambda i: (i, 0)),
            pl.BlockSpec((D, D), lambda i: (0, 0)),
            pl.BlockSpec((NC, RB, 16), lambda i: (0, i, 0)),
        ],
        out_specs=[
            pl.BlockSpec((RB, D), lambda i: (i, 0)),
            pl.BlockSpec((RB,), lambda i: (i,)),
        ],
        out_shape=[
            jax.ShapeDtypeStruct((N_NODES, D), jnp.float32),
            jax.ShapeDtypeStruct((N_NODES,), jnp.float32),
        ],
    )(x, W1, degp)


def _tc2_body(a0_ref, a1_ref, hs_ref, dis_ref, b_ref, w_ref, out_ref):
    dis = dis_ref[...]
    z = (dis[:, None] * (a0_ref[...] + a1_ref[...] - hs_ref[...])
         + b_ref[...][None, :])
    z = jnp.maximum(z, 0.0)
    h2 = jnp.dot(z, w_ref[...], preferred_element_type=jnp.float32)
    out_ref[...] = h2 * dis[:, None]


@jax.jit
def _tc2_call(a0, a1, hs1, dis, b1, W2):
    grid = (N_NODES // RB,)
    return pl.pallas_call(
        _tc2_body,
        grid=grid,
        in_specs=[
            pl.BlockSpec((RB, D), lambda i: (i, 0)),
            pl.BlockSpec((RB, D), lambda i: (i, 0)),
            pl.BlockSpec((RB, D), lambda i: (i, 0)),
            pl.BlockSpec((RB,), lambda i: (i,)),
            pl.BlockSpec((D,), lambda i: (0,)),
            pl.BlockSpec((D, D), lambda i: (0, 0)),
        ],
        out_specs=pl.BlockSpec((RB, D), lambda i: (i, 0)),
        out_shape=jax.ShapeDtypeStruct((N_NODES, D), jnp.float32),
    )(a0, a1, hs1, dis, b1, W2)


def _tc3_body(a0_ref, a1_ref, hs_ref, dis_ref, b_ref, out_ref):
    dis = dis_ref[...]
    out_ref[...] = (dis[:, None] * (a0_ref[...] + a1_ref[...] - hs_ref[...])
                    + b_ref[...][None, :])


@jax.jit
def _tc3_call(a0, a1, hs2, dis, b2):
    grid = (N_NODES // RB,)
    return pl.pallas_call(
        _tc3_body,
        grid=grid,
        in_specs=[
            pl.BlockSpec((RB, D), lambda i: (i, 0)),
            pl.BlockSpec((RB, D), lambda i: (i, 0)),
            pl.BlockSpec((RB, D), lambda i: (i, 0)),
            pl.BlockSpec((RB,), lambda i: (i,)),
            pl.BlockSpec((D,), lambda i: (0,)),
        ],
        out_specs=pl.BlockSpec((RB, D), lambda i: (i, 0)),
        out_shape=jax.ShapeDtypeStruct((N_NODES, D), jnp.float32),
    )(a0, a1, hs2, dis, b2)


def kernel(x, edge_index, edge_weight, W1, b1, W2, b2):
    src2 = edge_index[0].reshape(ROWS2D, CH)
    dst2 = edge_index[1].reshape(ROWS2D, CH)
    ew2 = edge_weight.reshape(ROWS2D, CH)

    degp = _deg_call(dst2, ew2)
    hs1, dis = _tc1_call(x, W1, degp)
    acc1 = _edge_call(hs1, src2, dst2, ew2)
    hs2 = _tc2_call(acc1[0], acc1[1], hs1, dis, b1, W2)
    acc2 = _edge_call(hs2, src2, dst2, ew2)
    out = _tc3_call(acc2[0], acc2[1], hs2, dis, b2)
    return out


# speedup vs baseline: 15.5917x; 15.5917x over previous
import jax
import jax.numpy as jnp
from jax import lax
from jax.experimental import pallas as pl
from jax.experimental.pallas import tpu as pltpu
from jax.experimental.pallas import tpu_sc as plsc

N_NODES = 10000
NP = 10240
E_EDGES = 320000
D = 128

NC = 2
NS = 16
NW = NC * NS
CH = 80
SB = 5
CPS = 25
NPT = NP // NS

_MESH = plsc.VectorSubcoreMesh(core_axis_name="c", subcore_axis_name="s")
_SC_PARAMS = pltpu.CompilerParams(needs_layout_passes=False)


def _deg_body(dst_hbm, ew_hbm, out_hbm, dst_v, ew_v, zer_v, acc_s, sem):
    cid = lax.axis_index("c")
    sid = lax.axis_index("s")
    wid = cid * NS + sid

    zv = jnp.zeros((16,), jnp.float32)

    def zrow(i, _):
        zer_v[pl.ds(i * 16, 16)] = zv
        return ()

    lax.fori_loop(0, NPT // 16, zrow, ())
    pltpu.sync_copy(zer_v, acc_s.at[pl.ds(sid * NPT, NPT)])
    plsc.subcore_barrier()

    def superblock(sb, _):
        pltpu.sync_copy(dst_hbm.at[wid, sb], dst_v)
        pltpu.sync_copy(ew_hbm.at[wid, sb], ew_v)

        def chunk(c, _):
            pltpu.sync_copy(ew_v.at[c], acc_s.at[dst_v.at[c]], add=True)
            return ()

        lax.fori_loop(0, CPS, chunk, ())
        return ()

    lax.fori_loop(0, SB, superblock, ())
    plsc.subcore_barrier()

    @pl.when(sid == 0)
    def _():
        pltpu.sync_copy(acc_s, out_hbm.at[cid])


@jax.jit
def _deg_call(dst4, ew4):
    return pl.kernel(
        _deg_body,
        out_type=jax.ShapeDtypeStruct((NC, NP), jnp.float32),
        mesh=_MESH,
        compiler_params=_SC_PARAMS,
        scratch_types=[
            pltpu.VMEM((CPS, CH), jnp.int32),
            pltpu.VMEM((CPS, CH), jnp.float32),
            pltpu.VMEM((NPT,), jnp.float32),
            pltpu.VMEM_SHARED((NP,), jnp.float32),
            pltpu.SemaphoreType.DMA,
        ],
    )(dst4, ew4)


def _edge_body(hs_hbm, src_hbm, dst_hbm, ew_hbm, out_hbm,
               src_v, dst_v, ew_v, rows_v, acc_s, sem):
    cid = lax.axis_index("c")
    sid = lax.axis_index("s")
    wid = cid * NS + sid
    nb = sid * NPT
    pltpu.sync_copy(hs_hbm.at[pl.ds(nb, NPT)], acc_s.at[pl.ds(nb, NPT)])
    plsc.subcore_barrier()

    def superblock(sb, _):
        pltpu.sync_copy(src_hbm.at[wid, sb], src_v)
        pltpu.sync_copy(dst_hbm.at[wid, sb], dst_v)
        pltpu.sync_copy(ew_hbm.at[wid, sb], ew_v)

        def chunk(c, _):
            pltpu.async_copy(hs_hbm.at[src_v.at[c]], rows_v, sem).wait()
            ewrow = ew_v.at[c]
            for g in range(CH // 16):
                ew16 = ewrow[pl.ds(g * 16, 16)]
                for l in range(16):
                    e = g * 16 + l
                    bc = ew16.at[jnp.full((16,), l, jnp.int32)].get(
                        mode="promise_in_bounds")
                    r = rows_v.at[e]
                    for j in range(D // 16):
                        r[pl.ds(j * 16, 16)] = r[pl.ds(j * 16, 16)] * bc
            pltpu.sync_copy(rows_v, acc_s.at[dst_v.at[c]], add=True)
            return ()

        lax.fori_loop(0, CPS, chunk, ())
        return ()

    lax.fori_loop(0, SB, superblock, ())
    plsc.subcore_barrier()
    pltpu.sync_copy(acc_s.at[pl.ds(nb, NPT)], out_hbm.at[cid, pl.ds(nb, NPT)])


@jax.jit
def _edge_call(hs, src4, dst4, ew4):
    return pl.kernel(
        _edge_body,
        out_type=jax.ShapeDtypeStruct((NC, NP, D), jnp.float32),
        mesh=_MESH,
        compiler_params=_SC_PARAMS,
        scratch_types=[
            pltpu.VMEM((CPS, CH), jnp.int32),
            pltpu.VMEM((CPS, CH), jnp.int32),
            pltpu.VMEM((CPS, CH), jnp.float32),
            pltpu.VMEM((CH, D), jnp.float32),
            pltpu.VMEM_SHARED((NP, D), jnp.float32),
            pltpu.SemaphoreType.DMA,
        ],
    )(hs, src4, dst4, ew4)


RB = 1024


def _tc1_body(x_ref, w_ref, degp_ref, hs_ref, dis_ref):
    deg = jnp.sum(degp_ref[...], axis=0) + 1.0
    dis = jnp.where(deg > 0.0, lax.rsqrt(deg), 0.0)
    h = jnp.dot(x_ref[...], w_ref[...], preferred_element_type=jnp.float32)
    hs_ref[...] = h * dis[:, None]
    dis_ref[...] = dis[:, None]


@jax.jit
def _tc1_call(x, W1, degp):
    grid = (NP // RB,)
    return pl.pallas_call(
        _tc1_body,
        grid=grid,
        in_specs=[
            pl.BlockSpec((RB, D), lambda i: (i, 0)),
            pl.BlockSpec((D, D), lambda i: (0, 0)),
            pl.BlockSpec((NC, RB), lambda i: (0, i)),
        ],
        out_specs=[
            pl.BlockSpec((RB, D), lambda i: (i, 0)),
            pl.BlockSpec((RB, 1), lambda i: (i, 0)),
        ],
        out_shape=[
            jax.ShapeDtypeStruct((NP, D), jnp.float32),
            jax.ShapeDtypeStruct((NP, 1), jnp.float32),
        ],
    )(x, W1, degp)


def _tc2_body(a0_ref, a1_ref, hs_ref, dis_ref, b_ref, w_ref, out_ref):
    dis = dis_ref[...]
    z = (dis * (a0_ref[...] + a1_ref[...] - hs_ref[...])
         + b_ref[...][None, :])
    z = jnp.maximum(z, 0.0)
    h2 = jnp.dot(z, w_ref[...], preferred_element_type=jnp.float32)
    out_ref[...] = h2 * dis


@jax.jit
def _tc2_call(a0, a1, hs1, dis, b1, W2):
    grid = (NP // RB,)
    return pl.pallas_call(
        _tc2_body,
        grid=grid,
        in_specs=[
            pl.BlockSpec((RB, D), lambda i: (i, 0)),
            pl.BlockSpec((RB, D), lambda i: (i, 0)),
            pl.BlockSpec((RB, D), lambda i: (i, 0)),
            pl.BlockSpec((RB, 1), lambda i: (i, 0)),
            pl.BlockSpec((D,), lambda i: (0,)),
            pl.BlockSpec((D, D), lambda i: (0, 0)),
        ],
        out_specs=pl.BlockSpec((RB, D), lambda i: (i, 0)),
        out_shape=jax.ShapeDtypeStruct((NP, D), jnp.float32),
    )(a0, a1, hs1, dis, b1, W2)


def _tc3_body(a0_ref, a1_ref, hs_ref, dis_ref, b_ref, out_ref):
    dis = dis_ref[...]
    out_ref[...] = (dis * (a0_ref[...] + a1_ref[...] - hs_ref[...])
                    + b_ref[...][None, :])


@jax.jit
def _tc3_call(a0, a1, hs2, dis, b2):
    grid = (NP // RB,)
    return pl.pallas_call(
        _tc3_body,
        grid=grid,
        in_specs=[
            pl.BlockSpec((RB, D), lambda i: (i, 0)),
            pl.BlockSpec((RB, D), lambda i: (i, 0)),
            pl.BlockSpec((RB, D), lambda i: (i, 0)),
            pl.BlockSpec((RB, 1), lambda i: (i, 0)),
            pl.BlockSpec((D,), lambda i: (0,)),
        ],
        out_specs=pl.BlockSpec((RB, D), lambda i: (i, 0)),
        out_shape=jax.ShapeDtypeStruct((NP, D), jnp.float32),
    )(a0, a1, hs2, dis, b2)


def kernel(x, edge_index, edge_weight, W1, b1, W2, b2):
    src4 = edge_index[0].reshape(NW, SB, CPS, CH)
    dst4 = edge_index[1].reshape(NW, SB, CPS, CH)
    ew4 = edge_weight.reshape(NW, SB, CPS, CH)
    xp = jnp.pad(x, ((0, NP - N_NODES), (0, 0)))

    degp = _deg_call(dst4, ew4)
    hs1, dis = _tc1_call(xp, W1, degp)
    acc1 = _edge_call(hs1, src4, dst4, ew4)
    hs2 = _tc2_call(acc1[0], acc1[1], hs1, dis, b1, W2)
    acc2 = _edge_call(hs2, src4, dst4, ew4)
    out = _tc3_call(acc2[0], acc2[1], hs2, dis, b2)
    return out[:N_NODES]


# speedup vs baseline: 16.7971x; 1.0773x over previous
import jax
import jax.numpy as jnp
from jax import lax
from jax.experimental import pallas as pl
from jax.experimental.pallas import tpu as pltpu
from jax.experimental.pallas import tpu_sc as plsc

N_NODES = 10000
NP = 10240
E_EDGES = 320000
D = 128

NC = 2
NS = 16
NW = NC * NS
CH = 80
SB = 3
CPS = 42
DSB = 6
DCPS = 21
EP = NW * SB * CPS * CH
NPT = NP // NS
TRIOS = CPS // 3

_MESH = plsc.VectorSubcoreMesh(core_axis_name="c", subcore_axis_name="s")
_SC_PARAMS = pltpu.CompilerParams(needs_layout_passes=False)


def _deg_body(dst_hbm, ew_hbm, out_hbm, dst_v, ew_v, zer_v, acc_s, sem):
    cid = lax.axis_index("c")
    sid = lax.axis_index("s")
    wid = cid * NS + sid

    zv = jnp.zeros((16,), jnp.float32)

    def zrow(i, _):
        zer_v[pl.ds(i * 16, 16)] = zv
        return ()

    lax.fori_loop(0, NPT // 16, zrow, ())
    pltpu.sync_copy(zer_v, acc_s.at[pl.ds(sid * NPT, NPT)])
    plsc.subcore_barrier()

    def superblock(sb, _):
        pltpu.sync_copy(dst_hbm.at[wid, sb], dst_v)
        pltpu.sync_copy(ew_hbm.at[wid, sb], ew_v)

        def chunk(c, _):
            pltpu.sync_copy(ew_v.at[c], acc_s.at[dst_v.at[c]], add=True)
            return ()

        lax.fori_loop(0, DCPS, chunk, ())
        return ()

    lax.fori_loop(0, DSB, superblock, ())
    plsc.subcore_barrier()

    @pl.when(sid == 0)
    def _():
        pltpu.sync_copy(acc_s, out_hbm.at[cid])


@jax.jit
def _deg_call(dst4, ew4):
    return pl.kernel(
        _deg_body,
        out_type=jax.ShapeDtypeStruct((NC, NP), jnp.float32),
        mesh=_MESH,
        compiler_params=_SC_PARAMS,
        scratch_types=[
            pltpu.VMEM((DCPS, CH), jnp.int32),
            pltpu.VMEM((DCPS, CH), jnp.float32),
            pltpu.VMEM((NPT,), jnp.float32),
            pltpu.VMEM_SHARED((NP,), jnp.float32),
            pltpu.SemaphoreType.DMA,
        ],
    )(dst4, ew4)


def _edge_body(hs_hbm, src_hbm, dst_hbm, ew_hbm, out_hbm,
               src_v, dst_v, ew_v, b0, b1, b2,
               g0, g1, g2, s0, s1, s2, acc_s):
    cid = lax.axis_index("c")
    sid = lax.axis_index("s")
    wid = cid * NS + sid
    nb = sid * NPT
    pltpu.sync_copy(hs_hbm.at[pl.ds(nb, NPT)], acc_s.at[pl.ds(nb, NPT)])
    plsc.subcore_barrier()

    bufs = (b0, b1, b2)
    gsems = (g0, g1, g2)
    ssems = (s0, s1, s2)

    def wait_gather(k):
        pltpu.make_async_copy(
            hs_hbm.at[pl.ds(0, CH)], bufs[k], gsems[k]).wait()

    def wait_scatter(k):
        pltpu.make_async_copy(
            bufs[k], acc_s.at[pl.ds(0, CH)], ssems[k]).wait()

    def issue_gather(k, c):
        pltpu.async_copy(hs_hbm.at[src_v.at[c]], bufs[k], gsems[k])

    def issue_scatter(k, c):
        pltpu.async_copy(bufs[k], acc_s.at[dst_v.at[c]], ssems[k], add=True)

    def scale(k, c):
        buf = bufs[k]
        ewrow = ew_v.at[c]

        def grp(g, _):
            ew16 = ewrow[pl.ds(g * 16, 16)]
            base = g * 16
            for l in range(16):
                bc = ew16.at[jnp.full((16,), l, jnp.int32)].get(
                    mode="promise_in_bounds")
                r = buf.at[base + l]
                for j in range(D // 16):
                    r[pl.ds(j * 16, 16)] = r[pl.ds(j * 16, 16)] * bc
            return ()

        lax.fori_loop(0, CH // 16, grp, ())

    def superblock(sb, _):
        pltpu.sync_copy(src_hbm.at[wid, sb], src_v)
        pltpu.sync_copy(dst_hbm.at[wid, sb], dst_v)
        pltpu.sync_copy(ew_hbm.at[wid, sb], ew_v)

        @pl.when(sb > 0)
        def _():
            wait_scatter(0)
            wait_scatter(1)
        issue_gather(0, 0)
        issue_gather(1, 1)

        def trio(t, _):
            c0 = t * 3
            wait_gather(0)
            scale(0, c0)
            issue_scatter(0, c0)

            @pl.when(sb + t > 0)
            def _():
                wait_scatter(2)
            issue_gather(2, c0 + 2)

            wait_gather(1)
            scale(1, c0 + 1)
            issue_scatter(1, c0 + 1)

            @pl.when(t < TRIOS - 1)
            def _():
                wait_scatter(0)
                issue_gather(0, c0 + 3)

            wait_gather(2)
            scale(2, c0 + 2)
            issue_scatter(2, c0 + 2)

            @pl.when(t < TRIOS - 1)
            def _():
                wait_scatter(1)
                issue_gather(1, c0 + 4)
            return ()

        lax.fori_loop(0, TRIOS, trio, ())
        return ()

    lax.fori_loop(0, SB, superblock, ())
    wait_scatter(0)
    wait_scatter(1)
    wait_scatter(2)
    plsc.subcore_barrier()
    pltpu.sync_copy(acc_s.at[pl.ds(nb, NPT)], out_hbm.at[cid, pl.ds(nb, NPT)])


@jax.jit
def _edge_call(hs, src4, dst4, ew4):
    return pl.kernel(
        _edge_body,
        out_type=jax.ShapeDtypeStruct((NC, NP, D), jnp.float32),
        mesh=_MESH,
        compiler_params=_SC_PARAMS,
        scratch_types=[
            pltpu.VMEM((CPS, CH), jnp.int32),
            pltpu.VMEM((CPS, CH), jnp.int32),
            pltpu.VMEM((CPS, CH), jnp.float32),
            pltpu.VMEM((CH, D), jnp.float32),
            pltpu.VMEM((CH, D), jnp.float32),
            pltpu.VMEM((CH, D), jnp.float32),
            pltpu.SemaphoreType.DMA,
            pltpu.SemaphoreType.DMA,
            pltpu.SemaphoreType.DMA,
            pltpu.SemaphoreType.DMA,
            pltpu.SemaphoreType.DMA,
            pltpu.SemaphoreType.DMA,
            pltpu.VMEM_SHARED((NP, D), jnp.float32),
        ],
    )(hs, src4, dst4, ew4)


RB = 1024


def _tc1_body(x_ref, w_ref, degp_ref, hs_ref, dis_ref):
    deg = jnp.sum(degp_ref[...], axis=0) + 1.0
    dis = jnp.where(deg > 0.0, lax.rsqrt(deg), 0.0)
    h = jnp.dot(x_ref[...], w_ref[...], preferred_element_type=jnp.float32)
    hs_ref[...] = h * dis[:, None]
    dis_ref[...] = dis[:, None]


@jax.jit
def _tc1_call(x, W1, degp):
    grid = (NP // RB,)
    return pl.pallas_call(
        _tc1_body,
        grid=grid,
        in_specs=[
            pl.BlockSpec((RB, D), lambda i: (i, 0)),
            pl.BlockSpec((D, D), lambda i: (0, 0)),
            pl.BlockSpec((NC, RB), lambda i: (0, i)),
        ],
        out_specs=[
            pl.BlockSpec((RB, D), lambda i: (i, 0)),
            pl.BlockSpec((RB, 1), lambda i: (i, 0)),
        ],
        out_shape=[
            jax.ShapeDtypeStruct((NP, D), jnp.float32),
            jax.ShapeDtypeStruct((NP, 1), jnp.float32),
        ],
    )(x, W1, degp)


def _tc2_body(a0_ref, a1_ref, hs_ref, dis_ref, b_ref, w_ref, out_ref):
    dis = dis_ref[...]
    z = (dis * (a0_ref[...] + a1_ref[...] - hs_ref[...])
         + b_ref[...][None, :])
    z = jnp.maximum(z, 0.0)
    h2 = jnp.dot(z, w_ref[...], preferred_element_type=jnp.float32)
    out_ref[...] = h2 * dis


@jax.jit
def _tc2_call(a0, a1, hs1, dis, b1, W2):
    grid = (NP // RB,)
    return pl.pallas_call(
        _tc2_body,
        grid=grid,
        in_specs=[
            pl.BlockSpec((RB, D), lambda i: (i, 0)),
            pl.BlockSpec((RB, D), lambda i: (i, 0)),
            pl.BlockSpec((RB, D), lambda i: (i, 0)),
            pl.BlockSpec((RB, 1), lambda i: (i, 0)),
            pl.BlockSpec((D,), lambda i: (0,)),
            pl.BlockSpec((D, D), lambda i: (0, 0)),
        ],
        out_specs=pl.BlockSpec((RB, D), lambda i: (i, 0)),
        out_shape=jax.ShapeDtypeStruct((NP, D), jnp.float32),
    )(a0, a1, hs1, dis, b1, W2)


def _tc3_body(a0_ref, a1_ref, hs_ref, dis_ref, b_ref, out_ref):
    dis = dis_ref[...]
    out_ref[...] = (dis * (a0_ref[...] + a1_ref[...] - hs_ref[...])
                    + b_ref[...][None, :])


@jax.jit
def _tc3_call(a0, a1, hs2, dis, b2):
    grid = (NP // RB,)
    return pl.pallas_call(
        _tc3_body,
        grid=grid,
        in_specs=[
            pl.BlockSpec((RB, D), lambda i: (i, 0)),
            pl.BlockSpec((RB, D), lambda i: (i, 0)),
            pl.BlockSpec((RB, D), lambda i: (i, 0)),
            pl.BlockSpec((RB, 1), lambda i: (i, 0)),
            pl.BlockSpec((D,), lambda i: (0,)),
        ],
        out_specs=pl.BlockSpec((RB, D), lambda i: (i, 0)),
        out_shape=jax.ShapeDtypeStruct((NP, D), jnp.float32),
    )(a0, a1, hs2, dis, b2)


def kernel(x, edge_index, edge_weight, W1, b1, W2, b2):
    pad = EP - E_EDGES
    srcp = jnp.pad(edge_index[0], (0, pad))
    dstp = jnp.pad(edge_index[1], (0, pad))
    ewp = jnp.pad(edge_weight, (0, pad))
    src4 = srcp.reshape(NW, SB, CPS, CH)
    dst4 = dstp.reshape(NW, SB, CPS, CH)
    ew4 = ewp.reshape(NW, SB, CPS, CH)
    dstd = dstp.reshape(NW, DSB, DCPS, CH)
    ewd = ewp.reshape(NW, DSB, DCPS, CH)
    xp = jnp.pad(x, ((0, NP - N_NODES), (0, 0)))

    degp = _deg_call(dstd, ewd)
    hs1, dis = _tc1_call(xp, W1, degp)
    acc1 = _edge_call(hs1, src4, dst4, ew4)
    hs2 = _tc2_call(acc1[0], acc1[1], hs1, dis, b1, W2)
    acc2 = _edge_call(hs2, src4, dst4, ew4)
    out = _tc3_call(acc2[0], acc2[1], hs2, dis, b2)
    return out[:N_NODES]
